# baseline (speedup 1.0000x reference)
import functools

import jax
import jax.numpy as jnp
from jax import lax
from jax.experimental import pallas as pl
from jax.experimental.pallas import tpu as pltpu
from jax.experimental.pallas import tpu_sc as plsc

N, E, D, G = 10000, 320000, 128, 64
NC, NS = 2, 16
NW = NC * NS
EPW = 10240
EP = NW * EPW
RS = N // 10
EPS = 1e-5


def _lrelu(x):
    return jnp.where(x >= 0, x, 0.01 * x)


def _mm_body(x_ref, w_ref, o_ref):
    o_ref[...] = jnp.dot(x_ref[...], w_ref[...],
                         preferred_element_type=jnp.float32)


def _matmul(x, w):
    return pl.pallas_call(
        _mm_body,
        out_shape=jax.ShapeDtypeStruct((x.shape[0], w.shape[1]),
                                       jnp.float32),
    )(x, w)


def _bn_mm_body(p_ref, b_ref, g_ref, be_ref, w_ref, x_ref, h_ref):
    agg = p_ref[0] + p_ref[1] + b_ref[...]
    m = jnp.mean(agg, axis=0, keepdims=True)
    v = jnp.mean((agg - m) ** 2, axis=0, keepdims=True)
    xi = _lrelu((agg - m) * lax.rsqrt(v + EPS) * g_ref[...] + be_ref[...])
    x_ref[...] = xi
    h_ref[...] = jnp.dot(xi, w_ref[...], preferred_element_type=jnp.float32)


def _bn_mm(p, b, g, be, w):
    f_out = w.shape[1]
    return pl.pallas_call(
        _bn_mm_body,
        out_shape=(jax.ShapeDtypeStruct(p.shape[1:], jnp.float32),
                   jax.ShapeDtypeStruct((p.shape[1], f_out), jnp.float32)),
    )(p, b.reshape(1, -1), g.reshape(1, -1), be.reshape(1, -1), w)


def _head_body(p3_ref, b3_ref, g3_ref, be3_ref, x1_ref, x2_ref, bat_ref,
               fc1w_ref, fc1b_ref, fc2w_ref, fc2b_ref, fc3w_ref, fc3b_ref,
               c1w_ref, c1b_ref, c2w_ref, c2b_ref, c3w_ref, c3b_ref,
               o_ref):
    agg = p3_ref[0] + p3_ref[1] + b3_ref[...]
    m = jnp.mean(agg, axis=0, keepdims=True)
    v = jnp.mean((agg - m) ** 2, axis=0, keepdims=True)
    x3 = _lrelu((agg - m) * lax.rsqrt(v + EPS) * g3_ref[...] + be3_ref[...])
    oh = (bat_ref[...] == lax.broadcasted_iota(jnp.int32, (G, N), 0)
          ).astype(jnp.float32)
    p1 = jnp.dot(oh, x1_ref[...], preferred_element_type=jnp.float32)
    p2 = jnp.dot(oh, x2_ref[...], preferred_element_type=jnp.float32)
    p3 = jnp.dot(oh, x3, preferred_element_type=jnp.float32)
    o1 = _lrelu(jnp.dot(p1, fc1w_ref[...],
                        preferred_element_type=jnp.float32) + fc1b_ref[...])
    o2 = _lrelu(jnp.dot(p2, fc2w_ref[...],
                        preferred_element_type=jnp.float32) + fc2b_ref[...])
    o3 = _lrelu(jnp.dot(p3, fc3w_ref[...],
                        preferred_element_type=jnp.float32) + fc3b_ref[...])
    out = jnp.concatenate([o1, o2, o3], axis=1)
    h = _lrelu(jnp.dot(out, c1w_ref[...],
                       preferred_element_type=jnp.float32) + c1b_ref[...])
    h = _lrelu(jnp.dot(h, c2w_ref[...],
                       preferred_element_type=jnp.float32) + c2b_ref[...])
    o_ref[...] = jnp.dot(h, c3w_ref[...],
                         preferred_element_type=jnp.float32) + c3b_ref[...]


def _head(p3, b3, g3, be3, x1, x2, batch, fc1_W, fc1_b, fc2_W, fc2_b,
          fc3_W, fc3_b, c1_W, c1_b, c2_W, c2_b, c3_W, c3_b):
    r = lambda a: a.reshape(1, -1)
    return pl.pallas_call(
        _head_body,
        out_shape=jax.ShapeDtypeStruct((G, 2), jnp.float32),
    )(p3, r(b3), r(g3), r(be3), x1, x2, batch.reshape(1, N),
      fc1_W, r(fc1_b), fc2_W, r(fc2_b), fc3_W, r(fc3_b),
      c1_W, r(c1_b), c2_W, r(c2_b), c3_W, r(c3_b))


def _make_edge_scatter(F):
    mesh = plsc.VectorSubcoreMesh(core_axis_name="c", subcore_axis_name="s")
    K = 512 if F <= 32 else 320
    K2 = K // 2
    WPW = EPW // K

    @functools.partial(
        pl.kernel,
        mesh=mesh,
        compiler_params=pltpu.CompilerParams(use_tc_tiling_on_sc=False),
        out_type=jax.ShapeDtypeStruct((NC, N, F), jnp.float32),
        scratch_types=[
            pltpu.VMEM_SHARED((N, F), jnp.float32),
            pltpu.VMEM_SHARED((N, F), jnp.float32),
            [pltpu.VMEM((K,), jnp.int32) for _ in range(4)],
            [pltpu.VMEM((K,), jnp.int32) for _ in range(4)],
            [pltpu.VMEM((K,), jnp.float32) for _ in range(4)],
            [pltpu.VMEM((K, F), jnp.float32) for _ in range(2)],
            [pltpu.SemaphoreType.DMA for _ in range(4)],
            [pltpu.SemaphoreType.DMA for _ in range(2)],
            [pltpu.SemaphoreType.DMA for _ in range(2)],
        ],
    )
    def scatter(h_hbm, src_hbm, dst_hbm, ew_hbm, zero_hbm, out_hbm,
                h_sh, acc_sh, idx_s, idx_d, ew_v, rows,
                sem_lin, sem_g, sem_s):
        c = lax.axis_index("c")
        s = lax.axis_index("s")
        wid = c * NS + s

        @pl.when(s < 10)
        def _stage():
            pltpu.sync_copy(h_hbm.at[pl.ds(s * RS, RS)],
                            h_sh.at[pl.ds(s * RS, RS)])
            pltpu.sync_copy(zero_hbm.at[pl.ds(s * RS, RS)],
                            acc_sh.at[pl.ds(s * RS, RS)])

        plsc.subcore_barrier()

        def _start_lin(w, l):
            e0 = jnp.minimum((wid * WPW + w) * K, E - K)
            pltpu.async_copy(src_hbm.at[pl.ds(e0, K)], idx_s[l], sem_lin[l])
            pltpu.async_copy(dst_hbm.at[pl.ds(e0, K)], idx_d[l], sem_lin[l])
            pltpu.async_copy(ew_hbm.at[pl.ds(e0, K)], ew_v[l], sem_lin[l])

        def _wait_lin(l):
            pltpu.make_async_copy(src_hbm.at[pl.ds(0, K)], idx_s[l],
                                  sem_lin[l]).wait()
            pltpu.make_async_copy(dst_hbm.at[pl.ds(0, K)], idx_d[l],
                                  sem_lin[l]).wait()
            pltpu.make_async_copy(ew_hbm.at[pl.ds(0, K)], ew_v[l],
                                  sem_lin[l]).wait()

        def _wait_scatter(b, l):
            pltpu.make_async_copy(rows[b], acc_sh.at[idx_d[l]],
                                  sem_s[b]).wait()

        def _compute(b, l, off, cnt, flag):
            def group(g16, carry2):
                k = off + g16 * 16
                ws = ew_v[l][pl.ds(k, 16)] * flag
                for j in range(16):
                    wv = ws[j]
                    for cc in range(F // 16):
                        sl = pl.ds(cc * 16, 16)
                        rows[b][k + j, sl] = rows[b][k + j, sl] * wv
                return carry2

            lax.fori_loop(0, cnt // 16, group, 0)

        _start_lin(0, 0)

        def quad(m, carry):
            w0 = m * 4
            for j in range(4):
                w = w0 + j
                b, l = j % 2, j

                @pl.when(w >= 2)
                def _(b=b, j=j):
                    _wait_scatter(b, (j + 2) % 4)

                @pl.when(w + 1 < WPW)
                def _(w=w, j=j):
                    _start_lin(w + 1, (j + 1) % 4)

                _wait_lin(l)
                flag = jnp.where((wid * WPW + w) * K < E,
                                 jnp.float32(1.0), jnp.float32(0.0))
                g0 = pltpu.async_copy(
                    h_sh.at[idx_s[l].at[pl.ds(0, K2)]],
                    rows[b].at[pl.ds(0, K2)], sem_g[0])
                g1 = pltpu.async_copy(
                    h_sh.at[idx_s[l].at[pl.ds(K2, K2)]],
                    rows[b].at[pl.ds(K2, K2)], sem_g[1])
                g0.wait()
                _compute(b, l, 0, K2, flag)
                g1.wait()
                _compute(b, l, K2, K2, flag)
                pltpu.async_copy(rows[b], acc_sh.at[idx_d[l]], sem_s[b],
                                 add=True)
            return carry

        lax.fori_loop(0, WPW // 4, quad, 0)
        _wait_scatter(0, 2)
        _wait_scatter(1, 3)
        plsc.subcore_barrier()

        @pl.when(s < 10)
        def _drain():
            pltpu.sync_copy(acc_sh.at[pl.ds(s * RS, RS)],
                            out_hbm.at[c, pl.ds(s * RS, RS)])

    return scatter


_scatter_cache = {}


def _edge_scatter(F, *args):
    if F not in _scatter_cache:
        _scatter_cache[F] = _make_edge_scatter(F)
    return _scatter_cache[F](*args)


def kernel(x, edge_index, edge_weigth, batch, W1, b1, W2, b2, W3, b3,
           g1, be1, g2, be2, g3, be3, fc1_W, fc1_b, fc2_W, fc2_b,
           fc3_W, fc3_b, c1_W, c1_b, c2_W, c2_b, c3_W, c3_b):
    z32 = jnp.zeros((N, 32), jnp.float32)
    z64 = jnp.zeros((N, 64), jnp.float32)
    src, dst = edge_index[0], edge_index[1]

    h1 = _matmul(x, W1)
    p1 = _edge_scatter(32, h1, src, dst, edge_weigth, z32)
    x1, h2 = _bn_mm(p1, b1, g1, be1, W2)
    p2 = _edge_scatter(32, h2, src, dst, edge_weigth, z32)
    x2, h3 = _bn_mm(p2, b2, g2, be2, W3)
    p3 = _edge_scatter(64, h3, src, dst, edge_weigth, z64)
    return _head(p3, b3, g3, be3, x1, x2, batch, fc1_W, fc1_b,
                 fc2_W, fc2_b, fc3_W, fc3_b, c1_W, c1_b, c2_W, c2_b,
                 c3_W, c3_b)


# speedup vs baseline: 1.2766x; 1.2766x over previous
import functools

import jax
import jax.numpy as jnp
from jax import lax
from jax.experimental import pallas as pl
from jax.experimental.pallas import tpu as pltpu
from jax.experimental.pallas import tpu_sc as plsc

N, E, D, G = 10000, 320000, 128, 64
NC, NS = 2, 16
NW = NC * NS
EPW = 10240
EP = NW * EPW
RS = N // 10
EPS = 1e-5


def _lrelu(x):
    return jnp.where(x >= 0, x, 0.01 * x)


def _mm_body(x_ref, w_ref, o_ref):
    o_ref[...] = jnp.dot(x_ref[...], w_ref[...],
                         preferred_element_type=jnp.float32)


def _matmul(x, w):
    return pl.pallas_call(
        _mm_body,
        out_shape=jax.ShapeDtypeStruct((x.shape[0], w.shape[1]),
                                       jnp.float32),
    )(x, w)


def _bn_mm_body(p_ref, b_ref, g_ref, be_ref, w_ref, x_ref, h_ref):
    agg = p_ref[0] + p_ref[1] + b_ref[...]
    m = jnp.mean(agg, axis=0, keepdims=True)
    v = jnp.mean((agg - m) ** 2, axis=0, keepdims=True)
    xi = _lrelu((agg - m) * lax.rsqrt(v + EPS) * g_ref[...] + be_ref[...])
    x_ref[...] = xi
    h_ref[...] = jnp.dot(xi, w_ref[...], preferred_element_type=jnp.float32)


def _bn_mm(p, b, g, be, w):
    f_out = w.shape[1]
    return pl.pallas_call(
        _bn_mm_body,
        out_shape=(jax.ShapeDtypeStruct(p.shape[1:], jnp.float32),
                   jax.ShapeDtypeStruct((p.shape[1], f_out), jnp.float32)),
    )(p, b.reshape(1, -1), g.reshape(1, -1), be.reshape(1, -1), w)


def _head_body(p3_ref, b3_ref, g3_ref, be3_ref, x1_ref, x2_ref, bat_ref,
               fc1w_ref, fc1b_ref, fc2w_ref, fc2b_ref, fc3w_ref, fc3b_ref,
               c1w_ref, c1b_ref, c2w_ref, c2b_ref, c3w_ref, c3b_ref,
               o_ref):
    agg = p3_ref[0] + p3_ref[1] + b3_ref[...]
    m = jnp.mean(agg, axis=0, keepdims=True)
    v = jnp.mean((agg - m) ** 2, axis=0, keepdims=True)
    x3 = _lrelu((agg - m) * lax.rsqrt(v + EPS) * g3_ref[...] + be3_ref[...])
    oh = (bat_ref[...] == lax.broadcasted_iota(jnp.int32, (G, N), 0)
          ).astype(jnp.float32)
    p1 = jnp.dot(oh, x1_ref[...], preferred_element_type=jnp.float32)
    p2 = jnp.dot(oh, x2_ref[...], preferred_element_type=jnp.float32)
    p3 = jnp.dot(oh, x3, preferred_element_type=jnp.float32)
    o1 = _lrelu(jnp.dot(p1, fc1w_ref[...],
                        preferred_element_type=jnp.float32) + fc1b_ref[...])
    o2 = _lrelu(jnp.dot(p2, fc2w_ref[...],
                        preferred_element_type=jnp.float32) + fc2b_ref[...])
    o3 = _lrelu(jnp.dot(p3, fc3w_ref[...],
                        preferred_element_type=jnp.float32) + fc3b_ref[...])
    out = jnp.concatenate([o1, o2, o3], axis=1)
    h = _lrelu(jnp.dot(out, c1w_ref[...],
                       preferred_element_type=jnp.float32) + c1b_ref[...])
    h = _lrelu(jnp.dot(h, c2w_ref[...],
                       preferred_element_type=jnp.float32) + c2b_ref[...])
    o_ref[...] = jnp.dot(h, c3w_ref[...],
                         preferred_element_type=jnp.float32) + c3b_ref[...]


def _head(p3, b3, g3, be3, x1, x2, batch, fc1_W, fc1_b, fc2_W, fc2_b,
          fc3_W, fc3_b, c1_W, c1_b, c2_W, c2_b, c3_W, c3_b):
    r = lambda a: a.reshape(1, -1)
    return pl.pallas_call(
        _head_body,
        out_shape=jax.ShapeDtypeStruct((G, 2), jnp.float32),
    )(p3, r(b3), r(g3), r(be3), x1, x2, batch.reshape(1, N),
      fc1_W, r(fc1_b), fc2_W, r(fc2_b), fc3_W, r(fc3_b),
      c1_W, r(c1_b), c2_W, r(c2_b), c3_W, r(c3_b))


def _make_edge_scatter(F):
    mesh = plsc.VectorSubcoreMesh(core_axis_name="c", subcore_axis_name="s")
    K = 512 if F <= 32 else 320
    K2 = K // 2
    WPW = EPW // K

    @functools.partial(
        pl.kernel,
        mesh=mesh,
        compiler_params=pltpu.CompilerParams(use_tc_tiling_on_sc=False),
        out_type=jax.ShapeDtypeStruct((NC, N, F), jnp.float32),
        scratch_types=[
            pltpu.VMEM_SHARED((N, F), jnp.float32),
            pltpu.VMEM_SHARED((N, F), jnp.float32),
            [pltpu.VMEM((K,), jnp.int32) for _ in range(4)],
            [pltpu.VMEM((K,), jnp.int32) for _ in range(4)],
            [pltpu.VMEM((K,), jnp.float32) for _ in range(4)],
            [pltpu.VMEM((K, F), jnp.float32) for _ in range(2)],
            [pltpu.SemaphoreType.DMA for _ in range(4)],
            [pltpu.SemaphoreType.DMA for _ in range(2)],
            [pltpu.SemaphoreType.DMA for _ in range(2)],
        ],
    )
    def scatter(h_hbm, src_hbm, dst_hbm, ew_hbm, zero_hbm, out_hbm,
                h_sh, acc_sh, idx_s, idx_d, ew_v, rows,
                sem_lin, sem_g, sem_s):
        c = lax.axis_index("c")
        s = lax.axis_index("s")
        wid = c * NS + s

        @pl.when(s < 10)
        def _stage():
            pltpu.sync_copy(h_hbm.at[pl.ds(s * RS, RS)],
                            h_sh.at[pl.ds(s * RS, RS)])
            pltpu.sync_copy(zero_hbm.at[pl.ds(s * RS, RS)],
                            acc_sh.at[pl.ds(s * RS, RS)])

        plsc.subcore_barrier()

        def _start_lin(w, l):
            e0 = (wid * WPW + w) * K
            pltpu.async_copy(src_hbm.at[pl.ds(e0, K)], idx_s[l], sem_lin[l])
            pltpu.async_copy(dst_hbm.at[pl.ds(e0, K)], idx_d[l], sem_lin[l])
            pltpu.async_copy(ew_hbm.at[pl.ds(e0, K)], ew_v[l], sem_lin[l])

        def _wait_lin(l):
            pltpu.make_async_copy(src_hbm.at[pl.ds(0, K)], idx_s[l],
                                  sem_lin[l]).wait()
            pltpu.make_async_copy(dst_hbm.at[pl.ds(0, K)], idx_d[l],
                                  sem_lin[l]).wait()
            pltpu.make_async_copy(ew_hbm.at[pl.ds(0, K)], ew_v[l],
                                  sem_lin[l]).wait()

        def _wait_scatter(b, l):
            pltpu.make_async_copy(rows[b], acc_sh.at[idx_d[l]],
                                  sem_s[b]).wait()

        def _compute(b, l, off, cnt):
            def group(g16, carry2):
                k = off + g16 * 16
                ws = ew_v[l][pl.ds(k, 16)]
                for j in range(16):
                    wv = ws[j]
                    for cc in range(F // 16):
                        sl = pl.ds(cc * 16, 16)
                        rows[b][k + j, sl] = rows[b][k + j, sl] * wv
                return carry2

            lax.fori_loop(0, cnt // 16, group, 0)

        _start_lin(0, 0)

        def quad(m, carry):
            w0 = m * 4
            for j in range(4):
                w = w0 + j
                b, l = j % 2, j

                @pl.when(w >= 2)
                def _(b=b, j=j):
                    _wait_scatter(b, (j + 2) % 4)

                @pl.when(w + 1 < WPW)
                def _(w=w, j=j):
                    _start_lin(w + 1, (j + 1) % 4)

                _wait_lin(l)
                g0 = pltpu.async_copy(
                    h_sh.at[idx_s[l].at[pl.ds(0, K2)]],
                    rows[b].at[pl.ds(0, K2)], sem_g[0])
                g1 = pltpu.async_copy(
                    h_sh.at[idx_s[l].at[pl.ds(K2, K2)]],
                    rows[b].at[pl.ds(K2, K2)], sem_g[1])
                g0.wait()
                _compute(b, l, 0, K2)
                g1.wait()
                _compute(b, l, K2, K2)
                pltpu.async_copy(rows[b], acc_sh.at[idx_d[l]], sem_s[b],
                                 add=True)
            return carry

        lax.fori_loop(0, WPW // 4, quad, 0)
        _wait_scatter(0, 2)
        _wait_scatter(1, 3)
        plsc.subcore_barrier()

        @pl.when(s < 10)
        def _drain():
            pltpu.sync_copy(acc_sh.at[pl.ds(s * RS, RS)],
                            out_hbm.at[c, pl.ds(s * RS, RS)])

    return scatter


_scatter_cache = {}


def _edge_scatter(F, *args):
    if F not in _scatter_cache:
        _scatter_cache[F] = _make_edge_scatter(F)
    return _scatter_cache[F](*args)


def kernel(x, edge_index, edge_weigth, batch, W1, b1, W2, b2, W3, b3,
           g1, be1, g2, be2, g3, be3, fc1_W, fc1_b, fc2_W, fc2_b,
           fc3_W, fc3_b, c1_W, c1_b, c2_W, c2_b, c3_W, c3_b):
    pad = EP - E
    src_p = jnp.concatenate([edge_index[0], jnp.zeros((pad,), jnp.int32)])
    dst_p = jnp.concatenate(
        [edge_index[1], jnp.arange(pad, dtype=jnp.int32)])
    ew_p = jnp.concatenate([edge_weigth, jnp.zeros((pad,), jnp.float32)])
    z32 = jnp.zeros((N, 32), jnp.float32)
    z64 = jnp.zeros((N, 64), jnp.float32)

    h1 = _matmul(x, W1)
    p1 = _edge_scatter(32, h1, src_p, dst_p, ew_p, z32)
    x1, h2 = _bn_mm(p1, b1, g1, be1, W2)
    p2 = _edge_scatter(32, h2, src_p, dst_p, ew_p, z32)
    x2, h3 = _bn_mm(p2, b2, g2, be2, W3)
    p3 = _edge_scatter(64, h3, src_p, dst_p, ew_p, z64)
    return _head(p3, b3, g3, be3, x1, x2, batch, fc1_W, fc1_b,
                 fc2_W, fc2_b, fc3_W, fc3_b, c1_W, c1_b, c2_W, c2_b,
                 c3_W, c3_b)
